# initial kernel scaffold (unmeasured)
import jax
import jax.numpy as jnp
from jax import lax
from jax.experimental import pallas as pl
from jax.experimental.pallas import tpu as pltpu


def kernel(x, Wq, K_ext, V_ext, Wo):
    B, Sq, Din = x.shape
    _, Skv_l, Hq, Dh = K_ext.shape
    Hl = Wq.shape[1] // Dh
    W = Hq // Hl
    Dout = Wo.shape[1]
    QB = 64
    NQB = Sq // QB

    bf16 = jnp.bfloat16
    xb = x.astype(bf16)
    Wqb = Wq.astype(bf16)
    Wob = Wo.astype(bf16)
    Kp = jnp.transpose(K_ext.astype(bf16), (2, 0, 1, 3)).reshape(
        W, Hl, B, Skv_l, Dh
    )
    Vp = jnp.transpose(V_ext.astype(bf16), (2, 0, 1, 3)).reshape(
        W, Hl, B, Skv_l, Dh
    )
    KVp = jnp.stack([Kp, Vp], axis=1)

    def body(
        x_ref, wq_ref, kvp_ref, wo_ref, out_ref,
        kv_buf, part_buf, q_ref, ctx_ref,
        kv_send_sems, kv_recv_sems, p_send_sems, p_recv_sems,
    ):
        me = lax.axis_index("i")

        kv_sends = []
        for d in range(1, W):
            dst = (me + d) % W
            rdma = pltpu.make_async_remote_copy(
                src_ref=kvp_ref.at[pl.ds(dst, 1)],
                dst_ref=kv_buf.at[pl.ds(d, 1)],
                send_sem=kv_send_sems.at[d],
                recv_sem=kv_recv_sems.at[d],
                device_id=(dst,),
                device_id_type=pl.DeviceIdType.MESH,
            )
            rdma.start()
            kv_sends.append(rdma)

        kv_buf[pl.ds(0, 1)] = kvp_ref[pl.ds(me, 1)]

        for b in range(B):
            q_ref[b] = jnp.dot(
                x_ref[b], wq_ref[...], preferred_element_type=jnp.float32
            ).astype(bf16)

        for r in kv_sends:
            r.wait_recv()
        for r in kv_sends:
            r.wait_send()

        for b in range(B):
            for h in range(Hl):
                for qb in range(NQB):
                    q = q_ref[b, qb * QB:(qb + 1) * QB, h * Dh:(h + 1) * Dh]
                    k = kv_buf[:, 0, h, b, qb * QB:(qb + 1) * QB, :].reshape(
                        W * QB, Dh
                    )
                    v = kv_buf[:, 1, h, b, qb * QB:(qb + 1) * QB, :].reshape(
                        W * QB, Dh
                    )
                    scores = lax.dot_general(
                        q, k, (((1,), (1,)), ((), ())),
                        preferred_element_type=jnp.float32,
                    ) * 0.125
                    m = jnp.max(scores, axis=-1, keepdims=True)
                    w = jnp.exp(scores - m)
                    w = (w / jnp.sum(w, axis=-1, keepdims=True)).astype(bf16)
                    ctx = lax.dot_general(
                        w, v, (((1,), (0,)), ((), ())),
                        preferred_element_type=jnp.float32,
                    ).astype(bf16)
                    ctx_ref[b, qb * QB:(qb + 1) * QB, h * Dh:(h + 1) * Dh] = ctx

        for b in range(B):
            part_buf[0, b] = jnp.dot(
                ctx_ref[b], wo_ref[...], preferred_element_type=jnp.float32
            ).astype(bf16)

        p_sends = []
        for d in range(1, W):
            dst = (me + d) % W
            rdma = pltpu.make_async_remote_copy(
                src_ref=part_buf.at[pl.ds(0, 1)],
                dst_ref=part_buf.at[pl.ds(d, 1)],
                send_sem=p_send_sems.at[d],
                recv_sem=p_recv_sems.at[d],
                device_id=(dst,),
                device_id_type=pl.DeviceIdType.MESH,
            )
            rdma.start()
            p_sends.append(rdma)
        for r in p_sends:
            r.wait_recv()
        for r in p_sends:
            r.wait_send()

        out_ref[...] = jnp.sum(part_buf[...].astype(jnp.float32), axis=0)

    return pl.pallas_call(
        body,
        out_shape=jax.ShapeDtypeStruct((B, Sq, Dout), jnp.float32),
        in_specs=[pl.BlockSpec(memory_space=pltpu.VMEM)] * 4,
        out_specs=pl.BlockSpec(memory_space=pltpu.VMEM),
        scratch_shapes=[
            pltpu.VMEM((W, 2, Hl, B, Skv_l, Dh), bf16),
            pltpu.VMEM((W, B, Sq, Dout), bf16),
            pltpu.VMEM((B, Sq, Hl * Dh), bf16),
            pltpu.VMEM((B, Sq, Hl * Dh), bf16),
            pltpu.SemaphoreType.DMA((W,)),
            pltpu.SemaphoreType.DMA((W,)),
            pltpu.SemaphoreType.DMA((W,)),
            pltpu.SemaphoreType.DMA((W,)),
        ],
        compiler_params=pltpu.CompilerParams(collective_id=0),
    )(xb, Wqb, KVp, Wob)


# baseline (device time: 375450 ns/iter reference)
import jax
import jax.numpy as jnp
from jax import lax
from jax.experimental import pallas as pl
from jax.experimental.pallas import tpu as pltpu


def kernel(x, Wq, K_ext, V_ext, Wo):
    B, Sq, Din = x.shape
    _, Skv_l, Hq, Dh = K_ext.shape
    Hl = Wq.shape[1] // Dh
    W = Hq // Hl
    Dout = Wo.shape[1]
    QB = 64
    NQB = Sq // QB

    bf16 = jnp.bfloat16
    xb = x.astype(bf16)
    Wqb = Wq.astype(bf16)
    Wob = Wo.astype(bf16)
    Kp = jnp.transpose(K_ext.astype(bf16), (2, 0, 1, 3)).reshape(
        W, Hl, B, Skv_l, Dh
    )
    Vp = jnp.transpose(V_ext.astype(bf16), (2, 0, 1, 3)).reshape(
        W, Hl, B, Skv_l, Dh
    )
    KVp = jnp.stack([Kp, Vp], axis=1)

    def body(
        x_ref, wq_ref, kvp_ref, wo_ref, out_ref,
        kv_buf, part_buf, q_ref, ctx_ref,
        kv_send_sems, kv_recv_sems, p_send_sems, p_recv_sems,
    ):
        me = lax.axis_index("i")

        kv_sends = []
        for d in range(1, W):
            dst = (me + d) % W
            rdma = pltpu.make_async_remote_copy(
                src_ref=kvp_ref.at[pl.ds(dst, 1)],
                dst_ref=kv_buf.at[pl.ds(d, 1)],
                send_sem=kv_send_sems.at[d],
                recv_sem=kv_recv_sems.at[d],
                device_id=(dst,),
                device_id_type=pl.DeviceIdType.MESH,
            )
            rdma.start()
            kv_sends.append(rdma)

        kv_buf[pl.ds(0, 1)] = kvp_ref[pl.ds(me, 1)]

        for b in range(B):
            q_ref[b] = jnp.dot(
                x_ref[b], wq_ref[...], preferred_element_type=jnp.float32
            ).astype(bf16)

        for r in kv_sends:
            r.wait_recv()
        for r in kv_sends:
            r.wait_send()

        for b in range(B):
            for h in range(Hl):
                for qb in range(NQB):
                    q = q_ref[b, qb * QB:(qb + 1) * QB, h * Dh:(h + 1) * Dh]
                    k = kv_buf[:, 0, h, b, qb * QB:(qb + 1) * QB, :].reshape(
                        W * QB, Dh
                    )
                    v = kv_buf[:, 1, h, b, qb * QB:(qb + 1) * QB, :].reshape(
                        W * QB, Dh
                    )
                    scores = lax.dot_general(
                        q, k, (((1,), (1,)), ((), ())),
                        preferred_element_type=jnp.float32,
                    ) * 0.125
                    m = jnp.max(scores, axis=-1, keepdims=True)
                    w = jnp.exp(scores - m)
                    w = (w / jnp.sum(w, axis=-1, keepdims=True)).astype(bf16)
                    ctx = lax.dot_general(
                        w, v, (((1,), (0,)), ((), ())),
                        preferred_element_type=jnp.float32,
                    ).astype(bf16)
                    ctx_ref[b, qb * QB:(qb + 1) * QB, h * Dh:(h + 1) * Dh] = ctx

        for b in range(B):
            part_buf[0, b] = jnp.dot(
                ctx_ref[b], wo_ref[...], preferred_element_type=jnp.float32
            ).astype(bf16)

        p_sends = []
        for d in range(1, W):
            dst = (me + d) % W
            rdma = pltpu.make_async_remote_copy(
                src_ref=part_buf.at[pl.ds(0, 1)],
                dst_ref=part_buf.at[pl.ds(d, 1)],
                send_sem=p_send_sems.at[d],
                recv_sem=p_recv_sems.at[d],
                device_id=(dst,),
                device_id_type=pl.DeviceIdType.MESH,
            )
            rdma.start()
            p_sends.append(rdma)
        for r in p_sends:
            r.wait_recv()
        for r in p_sends:
            r.wait_send()

        out_ref[...] = jnp.sum(part_buf[...].astype(jnp.float32), axis=0)

    return pl.pallas_call(
        body,
        out_shape=jax.ShapeDtypeStruct((B, Sq, Dout), jnp.float32),
        in_specs=[pl.BlockSpec(memory_space=pltpu.VMEM)] * 4,
        out_specs=pl.BlockSpec(memory_space=pltpu.VMEM),
        scratch_shapes=[
            pltpu.VMEM((W, 2, Hl, B, Skv_l, Dh), bf16),
            pltpu.VMEM((W, B, Sq, Dout), bf16),
            pltpu.VMEM((B, Sq, Hl * Dh), bf16),
            pltpu.VMEM((B, Sq, Hl * Dh), bf16),
            pltpu.SemaphoreType.DMA((W,)),
            pltpu.SemaphoreType.DMA((W,)),
            pltpu.SemaphoreType.DMA((W,)),
            pltpu.SemaphoreType.DMA((W,)),
        ],
    )(xb, Wqb, KVp, Wob)


# device time: 132625 ns/iter; 2.8309x vs baseline; 2.8309x over previous
import os

import jax
import jax.numpy as jnp
from jax import lax
from jax.experimental import pallas as pl
from jax.experimental.pallas import tpu as pltpu

_SKIP_A2A = os.environ.get("SKIP_A2A") == "1"
_SKIP_ATTN = os.environ.get("SKIP_ATTN") == "1"
_SKIP_AR = os.environ.get("SKIP_AR") == "1"


def kernel(x, Wq, K_ext, V_ext, Wo):
    B, Sq, Din = x.shape
    _, Skv_l, Hq, Dh = K_ext.shape
    Hl = Wq.shape[1] // Dh
    W = Hq // Hl
    Dout = Wo.shape[1]
    QB = 64
    NQB = Sq // QB

    bf16 = jnp.bfloat16
    xb = x.astype(bf16)
    Wqb = Wq.astype(bf16)
    Wob = Wo.astype(bf16)
    Kp = jnp.transpose(K_ext.astype(bf16), (2, 0, 1, 3)).reshape(
        W, Hl, B, Skv_l, Dh
    )
    Vp = jnp.transpose(V_ext.astype(bf16), (2, 0, 1, 3)).reshape(
        W, Hl, B, Skv_l, Dh
    )
    KVp = jnp.stack([Kp, Vp], axis=1)

    def body(
        x_ref, wq_ref, kvp_ref, wo_ref, out_ref,
        kv_buf, part_buf, q_ref, ctx_ref,
        kv_send_sems, kv_recv_sems, p_send_sems, p_recv_sems,
    ):
        me = lax.axis_index("i")

        kv_sends = []
        for d in range(1, W):
            if _SKIP_A2A:
                break
            dst = (me + d) % W
            rdma = pltpu.make_async_remote_copy(
                src_ref=kvp_ref.at[pl.ds(dst, 1)],
                dst_ref=kv_buf.at[pl.ds(d, 1)],
                send_sem=kv_send_sems.at[d],
                recv_sem=kv_recv_sems.at[d],
                device_id=(dst,),
                device_id_type=pl.DeviceIdType.MESH,
            )
            rdma.start()
            kv_sends.append(rdma)

        kv_buf[pl.ds(0, 1)] = kvp_ref[pl.ds(me, 1)]

        for b in range(B):
            q_ref[b] = jnp.dot(
                x_ref[b], wq_ref[...], preferred_element_type=jnp.float32
            ).astype(bf16)

        for r in kv_sends:
            r.wait_recv()
        for r in kv_sends:
            r.wait_send()

        if _SKIP_ATTN:
            ctx_ref[...] = q_ref[...]
        for b in range(B if not _SKIP_ATTN else 0):
            for h in range(Hl):
                for qb in range(NQB):
                    q = q_ref[b, qb * QB:(qb + 1) * QB, h * Dh:(h + 1) * Dh]
                    k = kv_buf[:, 0, h, b, qb * QB:(qb + 1) * QB, :].reshape(
                        W * QB, Dh
                    )
                    v = kv_buf[:, 1, h, b, qb * QB:(qb + 1) * QB, :].reshape(
                        W * QB, Dh
                    )
                    scores = lax.dot_general(
                        q, k, (((1,), (1,)), ((), ())),
                        preferred_element_type=jnp.float32,
                    ) * 0.125
                    m = jnp.max(scores, axis=-1, keepdims=True)
                    w = jnp.exp(scores - m)
                    w = (w / jnp.sum(w, axis=-1, keepdims=True)).astype(bf16)
                    ctx = lax.dot_general(
                        w, v, (((1,), (0,)), ((), ())),
                        preferred_element_type=jnp.float32,
                    ).astype(bf16)
                    ctx_ref[b, qb * QB:(qb + 1) * QB, h * Dh:(h + 1) * Dh] = ctx

        for b in range(B):
            part_buf[0, b] = jnp.dot(
                ctx_ref[b], wo_ref[...], preferred_element_type=jnp.float32
            ).astype(bf16)

        p_sends = []
        for d in range(1, W):
            if _SKIP_AR:
                break
            dst = (me + d) % W
            rdma = pltpu.make_async_remote_copy(
                src_ref=part_buf.at[pl.ds(0, 1)],
                dst_ref=part_buf.at[pl.ds(d, 1)],
                send_sem=p_send_sems.at[d],
                recv_sem=p_recv_sems.at[d],
                device_id=(dst,),
                device_id_type=pl.DeviceIdType.MESH,
            )
            rdma.start()
            p_sends.append(rdma)
        for r in p_sends:
            r.wait_recv()
        for r in p_sends:
            r.wait_send()

        out_ref[...] = jnp.sum(part_buf[...].astype(jnp.float32), axis=0)

    return pl.pallas_call(
        body,
        out_shape=jax.ShapeDtypeStruct((B, Sq, Dout), jnp.float32),
        in_specs=[pl.BlockSpec(memory_space=pltpu.VMEM)] * 4,
        out_specs=pl.BlockSpec(memory_space=pltpu.VMEM),
        scratch_shapes=[
            pltpu.VMEM((W, 2, Hl, B, Skv_l, Dh), bf16),
            pltpu.VMEM((W, B, Sq, Dout), bf16),
            pltpu.VMEM((B, Sq, Hl * Dh), bf16),
            pltpu.VMEM((B, Sq, Hl * Dh), bf16),
            pltpu.SemaphoreType.DMA((W,)),
            pltpu.SemaphoreType.DMA((W,)),
            pltpu.SemaphoreType.DMA((W,)),
            pltpu.SemaphoreType.DMA((W,)),
        ],
    )(xb, Wqb, KVp, Wob)
